# baseline (device time: 40830 ns/iter reference)
import jax
import jax.numpy as jnp
from jax import lax
from jax.experimental import pallas as pl
from jax.experimental.pallas import tpu as pltpu

N_DEV = 8
M_BLK = 512
K_BLK = 512
N_OUT = 2048

F8 = jnp.float8_e4m3fn


def kernel(x, w_mat, scale_x, scale_w):
    if x.dtype != F8:
        x = x.astype(F8)
    if w_mat.dtype != F8:
        w_mat = w_mat.astype(F8)

    def body(x_ref, w_ref, sx_ref, sw_ref, out_ref, buf_ref, send_sems, recv_sems):
        my = lax.axis_index("i")

        barrier = pltpu.get_barrier_semaphore()
        for k in range(1, N_DEV):
            pl.semaphore_signal(
                barrier, inc=1,
                device_id=((my + k) % N_DEV,),
                device_id_type=pl.DeviceIdType.MESH,
            )
        pl.semaphore_wait(barrier, N_DEV - 1)

        sends = []
        for k in range(1, N_DEV):
            e = (my + k) % N_DEV
            rdma = pltpu.make_async_remote_copy(
                src_ref=x_ref.at[pl.ds(e * M_BLK, M_BLK), :],
                dst_ref=buf_ref.at[my],
                send_sem=send_sems.at[k - 1],
                recv_sem=recv_sems.at[my],
                device_id=(e,),
                device_id_type=pl.DeviceIdType.MESH,
            )
            rdma.start()
            sends.append(rdma)

        acc = jnp.dot(
            x_ref[pl.ds(my * M_BLK, M_BLK), :],
            w_ref[pl.ds(my * K_BLK, K_BLK), :],
            preferred_element_type=jnp.float32,
        )

        for k in range(1, N_DEV):
            s = (my - k) % N_DEV
            recv = pltpu.make_async_remote_copy(
                src_ref=buf_ref.at[s],
                dst_ref=buf_ref.at[s],
                send_sem=send_sems.at[N_DEV - 1],
                recv_sem=recv_sems.at[s],
                device_id=(s,),
                device_id_type=pl.DeviceIdType.MESH,
            )
            recv.wait_recv()
            acc += jnp.dot(
                buf_ref[s],
                w_ref[pl.ds(s * K_BLK, K_BLK), :],
                preferred_element_type=jnp.float32,
            )

        for rdma in sends:
            rdma.wait_send()

        scale = sx_ref[0] * sw_ref[0]
        out_ref[:, :] = jnp.maximum(acc * scale, 0.0)

    return pl.pallas_call(
        body,
        out_shape=jax.ShapeDtypeStruct((M_BLK, N_OUT), jnp.float32),
        in_specs=[
            pl.BlockSpec(memory_space=pltpu.VMEM),
            pl.BlockSpec(memory_space=pltpu.VMEM),
            pl.BlockSpec(memory_space=pltpu.SMEM),
            pl.BlockSpec(memory_space=pltpu.SMEM),
        ],
        out_specs=pl.BlockSpec(memory_space=pltpu.VMEM),
        scratch_shapes=[
            pltpu.VMEM((N_DEV, M_BLK, K_BLK), F8),
            pltpu.SemaphoreType.DMA((N_DEV,)),
            pltpu.SemaphoreType.DMA((N_DEV,)),
        ],
        compiler_params=pltpu.CompilerParams(collective_id=0),
    )(x, w_mat, scale_x, scale_w)


# device time: 33951 ns/iter; 1.2026x vs baseline; 1.2026x over previous
import jax
import jax.numpy as jnp
from jax import lax
from jax.experimental import pallas as pl
from jax.experimental.pallas import tpu as pltpu

N_DEV = 8
M_BLK = 512
K_BLK = 512
N_OUT = 2048

F8 = jnp.float8_e4m3fn


def kernel(x, w_mat, scale_x, scale_w):
    def body(x_ref, w_hbm, sx_ref, sw_ref, out_ref,
             x8_ref, buf_ref, wbuf_ref, send_sems, recv_sems, wdma_sems):
        my = lax.axis_index("i")

        def w_copy(s, slot):
            return pltpu.make_async_copy(
                w_hbm.at[pl.ds(s * K_BLK, K_BLK), :],
                wbuf_ref.at[slot],
                wdma_sems.at[slot],
            )

        w_copy(my, 0).start()

        barrier = pltpu.get_barrier_semaphore()
        for k in range(1, N_DEV):
            pl.semaphore_signal(
                barrier, inc=1,
                device_id=((my + k) % N_DEV,),
                device_id_type=pl.DeviceIdType.MESH,
            )
        pl.semaphore_wait(barrier, N_DEV - 1)

        sends = []
        for k in range(1, N_DEV):
            e = (my + k) % N_DEV
            x8_ref[pl.ds(e * M_BLK, M_BLK), :] = (
                x_ref[pl.ds(e * M_BLK, M_BLK), :].astype(F8))
            rdma = pltpu.make_async_remote_copy(
                src_ref=x8_ref.at[pl.ds(e * M_BLK, M_BLK), :],
                dst_ref=buf_ref.at[my],
                send_sem=send_sems.at[k - 1],
                recv_sem=recv_sems.at[my],
                device_id=(e,),
                device_id_type=pl.DeviceIdType.MESH,
            )
            rdma.start()
            sends.append(rdma)
        x8_ref[pl.ds(my * M_BLK, M_BLK), :] = (
            x_ref[pl.ds(my * M_BLK, M_BLK), :].astype(F8))

        acc = None
        for j in range(N_DEV):
            s = my if j == 0 else (my - j) % N_DEV
            if j + 1 < N_DEV:
                w_copy((my - (j + 1)) % N_DEV, (j + 1) % 2).start()
            w_copy(s, j % 2).wait()
            w8 = wbuf_ref[j % 2].astype(F8)
            if j == 0:
                lhs = x8_ref[pl.ds(my * M_BLK, M_BLK), :]
            else:
                recv = pltpu.make_async_remote_copy(
                    src_ref=buf_ref.at[s],
                    dst_ref=buf_ref.at[s],
                    send_sem=send_sems.at[N_DEV - 1],
                    recv_sem=recv_sems.at[s],
                    device_id=(s,),
                    device_id_type=pl.DeviceIdType.MESH,
                )
                recv.wait_recv()
                lhs = buf_ref[s]
            d = jnp.dot(lhs, w8, preferred_element_type=jnp.float32)
            acc = d if acc is None else acc + d

        for rdma in sends:
            rdma.wait_send()

        scale = sx_ref[0] * sw_ref[0]
        out_ref[:, :] = jnp.maximum(acc * scale, 0.0)

    return pl.pallas_call(
        body,
        out_shape=jax.ShapeDtypeStruct((M_BLK, N_OUT), jnp.float32),
        in_specs=[
            pl.BlockSpec(memory_space=pltpu.VMEM),
            pl.BlockSpec(memory_space=pl.ANY),
            pl.BlockSpec(memory_space=pltpu.SMEM),
            pl.BlockSpec(memory_space=pltpu.SMEM),
        ],
        out_specs=pl.BlockSpec(memory_space=pltpu.VMEM),
        scratch_shapes=[
            pltpu.VMEM((N_DEV * M_BLK, K_BLK), F8),
            pltpu.VMEM((N_DEV, M_BLK, K_BLK), F8),
            pltpu.VMEM((2, K_BLK, N_OUT), jnp.float32),
            pltpu.SemaphoreType.DMA((N_DEV,)),
            pltpu.SemaphoreType.DMA((N_DEV,)),
            pltpu.SemaphoreType.DMA((2,)),
        ],
        compiler_params=pltpu.CompilerParams(
            collective_id=0, vmem_limit_bytes=100 * 1024 * 1024),
    )(x, w_mat, scale_x, scale_w)


# device time: 33093 ns/iter; 1.2338x vs baseline; 1.0259x over previous
import jax
import jax.numpy as jnp
from jax import lax
from jax.experimental import pallas as pl
from jax.experimental.pallas import tpu as pltpu

N_DEV = 8
M_BLK = 512
K_BLK = 512
N_OUT = 2048

F8 = jnp.float8_e4m3fn


def kernel(x, w_mat, scale_x, scale_w):
    def body(x_ref, w_hbm, sx_ref, sw_ref, out_ref,
             x8_ref, buf_ref, wbuf_ref, w8_ref, send_sems, recv_sems,
             wdma_sems):
        my = lax.axis_index("i")

        def w_copy(s, slot):
            return pltpu.make_async_copy(
                w_hbm.at[pl.ds(s * K_BLK, K_BLK), :],
                wbuf_ref.at[slot],
                wdma_sems.at[slot],
            )

        w_copy(my, 0).start()

        barrier = pltpu.get_barrier_semaphore()
        for k in range(1, N_DEV):
            pl.semaphore_signal(
                barrier, inc=1,
                device_id=((my + k) % N_DEV,),
                device_id_type=pl.DeviceIdType.MESH,
            )
        pl.semaphore_wait(barrier, N_DEV - 1)

        sends = []
        for k in range(1, N_DEV):
            e = (my + k) % N_DEV
            x8_ref[pl.ds(e * M_BLK, M_BLK), :] = (
                x_ref[pl.ds(e * M_BLK, M_BLK), :].astype(F8))
            rdma = pltpu.make_async_remote_copy(
                src_ref=x8_ref.at[pl.ds(e * M_BLK, M_BLK), :],
                dst_ref=buf_ref.at[my],
                send_sem=send_sems.at[k - 1],
                recv_sem=recv_sems.at[my],
                device_id=(e,),
                device_id_type=pl.DeviceIdType.MESH,
            )
            rdma.start()
            sends.append(rdma)
        x8_ref[pl.ds(my * M_BLK, M_BLK), :] = (
            x_ref[pl.ds(my * M_BLK, M_BLK), :].astype(F8))

        w_copy(my, 0).wait()
        w8_ref[my] = wbuf_ref[0].astype(F8)
        w_copy((my - 1) % N_DEV, 1).start()
        acc = jnp.dot(
            x8_ref[pl.ds(my * M_BLK, M_BLK), :],
            w8_ref[my],
            preferred_element_type=jnp.float32,
        )
        for j in range(1, N_DEV):
            s = (my - j) % N_DEV
            if j + 1 < N_DEV:
                w_copy((my - (j + 1)) % N_DEV, (j + 1) % 2).start()
            w_copy(s, j % 2).wait()
            w8_ref[s] = wbuf_ref[j % 2].astype(F8)

        for j in range(1, N_DEV):
            s = (my - j) % N_DEV
            recv = pltpu.make_async_remote_copy(
                src_ref=buf_ref.at[s],
                dst_ref=buf_ref.at[s],
                send_sem=send_sems.at[N_DEV - 1],
                recv_sem=recv_sems.at[s],
                device_id=(s,),
                device_id_type=pl.DeviceIdType.MESH,
            )
            recv.wait_recv()
            acc += jnp.dot(
                buf_ref[s], w8_ref[s], preferred_element_type=jnp.float32)

        for rdma in sends:
            rdma.wait_send()

        scale = sx_ref[0] * sw_ref[0]
        out_ref[:, :] = jnp.maximum(acc * scale, 0.0)

    return pl.pallas_call(
        body,
        out_shape=jax.ShapeDtypeStruct((M_BLK, N_OUT), jnp.float32),
        in_specs=[
            pl.BlockSpec(memory_space=pltpu.VMEM),
            pl.BlockSpec(memory_space=pl.ANY),
            pl.BlockSpec(memory_space=pltpu.SMEM),
            pl.BlockSpec(memory_space=pltpu.SMEM),
        ],
        out_specs=pl.BlockSpec(memory_space=pltpu.VMEM),
        scratch_shapes=[
            pltpu.VMEM((N_DEV * M_BLK, K_BLK), F8),
            pltpu.VMEM((N_DEV, M_BLK, K_BLK), F8),
            pltpu.VMEM((2, K_BLK, N_OUT), jnp.float32),
            pltpu.VMEM((N_DEV, K_BLK, N_OUT), F8),
            pltpu.SemaphoreType.DMA((N_DEV,)),
            pltpu.SemaphoreType.DMA((N_DEV,)),
            pltpu.SemaphoreType.DMA((2,)),
        ],
        compiler_params=pltpu.CompilerParams(
            collective_id=0, vmem_limit_bytes=100 * 1024 * 1024),
    )(x, w_mat, scale_x, scale_w)
